# baseline (speedup 1.0000x reference)
import functools

import jax
import jax.numpy as jnp
from jax import lax
from jax.experimental import pallas as pl
from jax.experimental.pallas import tpu as pltpu
from jax.experimental.pallas import tpu_sc as plsc

_BM = 256
_BN = 2048
_PREC = jax.lax.Precision.DEFAULT

_NC = 2
_NS = 16
_NW = _NC * _NS


def _dist_argmin_body(zb16_ref, z_ref, ct_ref, idx_ref, loss_ref,
                      runmin_ref, runidx_ref, truemin_ref):
    i = pl.program_id(0)
    j = pl.program_id(1)
    zb16 = zb16_ref[...]
    zb = z_ref[...]
    cb = ct_ref[...]
    m = lax.dot_general(zb16, cb, (((1,), (0,)), ((), ())),
                        preferred_element_type=jnp.float32)
    zsq = jnp.sum(zb * zb, axis=1, keepdims=True)
    csq = jnp.sum(cb * cb, axis=0, keepdims=True)
    d = (zsq + csq) - 2.0 * m
    bmin = jnp.min(d, axis=1, keepdims=True)
    col = lax.broadcasted_iota(jnp.int32, d.shape, 1)
    bidx = jnp.min(jnp.where(d == bmin, col, jnp.int32(2**30)),
                   axis=1, keepdims=True) + j * _BN
    bmin_bf = bmin.astype(jnp.bfloat16).astype(jnp.float32)

    @pl.when(j == 0)
    def _init():
        runmin_ref[...] = bmin_bf
        runidx_ref[...] = bidx
        truemin_ref[...] = bmin

    @pl.when(j > 0)
    def _update():
        better = bmin < runmin_ref[...]
        runidx_ref[...] = jnp.where(better, bidx, runidx_ref[...])
        runmin_ref[...] = jnp.where(better, bmin_bf, runmin_ref[...])
        truemin_ref[...] = jnp.minimum(truemin_ref[...], bmin)

    @pl.when(j == pl.num_programs(1) - 1)
    def _emit():
        idx_ref[...] = runidx_ref[...].reshape(-1)
        s = jnp.sum(truemin_ref[...])

        @pl.when(i == 0)
        def _first():
            loss_ref[0, 0] = s

        @pl.when(i > 0)
        def _acc():
            loss_ref[0, 0] = loss_ref[0, 0] + s


def _dist_argmin(z_flat, ct, interpret=False):
    n, k = z_flat.shape
    nk = ct.shape[1]
    grid = (n // _BM, nk // _BN)
    zb16 = z_flat.astype(jnp.bfloat16)
    return pl.pallas_call(
        _dist_argmin_body,
        grid=grid,
        in_specs=[
            pl.BlockSpec((_BM, k), lambda i, j: (i, 0)),
            pl.BlockSpec((_BM, k), lambda i, j: (i, 0)),
            pl.BlockSpec((k, _BN), lambda i, j: (0, j)),
        ],
        out_specs=[
            pl.BlockSpec((_BM,), lambda i, j: (i,)),
            pl.BlockSpec(memory_space=pltpu.SMEM),
        ],
        out_shape=[
            jax.ShapeDtypeStruct((n,), jnp.int32),
            jax.ShapeDtypeStruct((1, 1), jnp.float32),
        ],
        scratch_shapes=[
            pltpu.VMEM((_BM, 1), jnp.float32),
            pltpu.VMEM((_BM, 1), jnp.int32),
            pltpu.VMEM((_BM, 1), jnp.float32),
        ],
        compiler_params=pltpu.CompilerParams(
            dimension_semantics=("arbitrary", "arbitrary")),
        interpret=interpret,
    )(zb16, z_flat, ct)


def _sc_gather(codebook_padded, idx):
    bn = idx.shape[0]
    d = codebook_padded.shape[1]
    b_per_w = bn // _NW
    nchunk = b_per_w // 128
    idx_r = idx.reshape(_NW, nchunk, 128)
    mesh = plsc.VectorSubcoreMesh(core_axis_name="c", subcore_axis_name="s")

    @functools.partial(
        pl.kernel, mesh=mesh,
        out_type=jax.ShapeDtypeStruct((bn, d), jnp.float32),
        scratch_types=[
            pltpu.VMEM((nchunk, 128), jnp.int32),
            pltpu.VMEM((b_per_w, d), jnp.float32),
            pltpu.SemaphoreType.DMA,
        ],
    )
    def gather_k(table_hbm, idx_hbm, out_hbm, idx_v, rows_v, sem):
        wid = lax.axis_index("s") * _NC + lax.axis_index("c")
        base = wid * b_per_w
        pltpu.sync_copy(idx_hbm.at[wid], idx_v)
        cps = [pltpu.async_copy(table_hbm.at[idx_v.at[c]],
                                rows_v.at[pl.ds(c * 128, 128)], sem)
               for c in range(nchunk)]
        for cp in cps:
            cp.wait()
        pltpu.sync_copy(rows_v, out_hbm.at[pl.ds(base, b_per_w)])

    return gather_k(codebook_padded, idx_r)


def kernel(z, codebook):
    b, c, h, w = z.shape
    zp = jnp.transpose(z, (0, 2, 3, 1))
    z_flat = zp.reshape(-1, c)
    ct = codebook.T
    idx, loss_sum = _dist_argmin(z_flat, ct)
    cb_pad = jnp.pad(codebook, ((0, 0), (0, 128 - c)))
    q_flat = _sc_gather(cb_pad, idx)[:, :c]
    quantized = q_flat.reshape(zp.shape)
    mse = loss_sum[0, 0] / jnp.float32(z.size)
    total_loss = mse + 0.25 * mse
    quantized_st = zp + (quantized - zp)
    quantized_out = jnp.transpose(quantized_st, (0, 3, 1, 2))
    return (jnp.reshape(total_loss, ()), quantized_out, idx)
